# baseline (speedup 1.0000x reference)
import functools

import jax
import jax.numpy as jnp
from jax import lax
from jax.experimental import pallas as pl
from jax.experimental.pallas import tpu as pltpu
from jax.experimental.pallas import tpu_sc as plsc

N_NODES = 10000
D = 128
N_PAD = 10240
TRASH = 10200
DW = 144
NC, NS = 2, 16
NW = NC * NS
CHUNK = 64
G = 2
NGRP = 82
SG = NGRP // 2
E_PAD = NW * NGRP * G * CHUNK
E_TOT = 320000 + N_NODES
BLK = 1280


def _mm2_body(x_ref, wl_ref, wr_ref, xl_ref, xr_ref):
    x = x_ref[...]
    ml = jnp.dot(x, wl_ref[...], preferred_element_type=jnp.float32)
    col = lax.broadcasted_iota(jnp.int32, (BLK, DW), 1)
    xl_ref[...] = jnp.where(col == D, 1.0, jnp.pad(ml, ((0, 0), (0, DW - D))))
    xr_ref[...] = jnp.dot(x, wr_ref[...], preferred_element_type=jnp.float32)


def _mm2(x, wl, wr):
    n = x.shape[0]
    return pl.pallas_call(
        _mm2_body,
        grid=(n // BLK,),
        in_specs=[pl.BlockSpec((BLK, D), lambda i: (i, 0)),
                  pl.BlockSpec((D, D), lambda i: (0, 0)),
                  pl.BlockSpec((D, D), lambda i: (0, 0))],
        out_specs=(pl.BlockSpec((BLK, DW), lambda i: (i, 0)),
                   pl.BlockSpec((BLK, D), lambda i: (i, 0))),
        out_shape=(jax.ShapeDtypeStruct((n, DW), jnp.float32),
                   jax.ShapeDtypeStruct((n, D), jnp.float32)),
    )(x, wl, wr)


def _mid_body(a0_ref, a1_ref, b_ref, wl_ref, wr_ref, xl_ref, xr_ref):
    i = pl.program_id(0)
    v = a0_ref[...] + a1_ref[...]
    num = v[:, :D]
    den = v[:, D:D + 1]
    x = jnp.maximum(num / (den + 1e-16) + b_ref[...], 0.0)
    rows = i * BLK + lax.broadcasted_iota(jnp.int32, x.shape, 0)
    x = jnp.where(rows < N_NODES, x, 0.0)
    ml = jnp.dot(x, wl_ref[...], preferred_element_type=jnp.float32)
    col = lax.broadcasted_iota(jnp.int32, (BLK, DW), 1)
    xl_ref[...] = jnp.where(col == D, 1.0, jnp.pad(ml, ((0, 0), (0, DW - D))))
    xr_ref[...] = jnp.dot(x, wr_ref[...], preferred_element_type=jnp.float32)


def _mid(a0, a1, b, wl, wr):
    return pl.pallas_call(
        _mid_body,
        grid=(N_PAD // BLK,),
        in_specs=[pl.BlockSpec((BLK, DW), lambda i: (i, 0)),
                  pl.BlockSpec((BLK, DW), lambda i: (i, 0)),
                  pl.BlockSpec((D,), lambda i: (0,)),
                  pl.BlockSpec((D, D), lambda i: (0, 0)),
                  pl.BlockSpec((D, D), lambda i: (0, 0))],
        out_specs=(pl.BlockSpec((BLK, DW), lambda i: (i, 0)),
                   pl.BlockSpec((BLK, D), lambda i: (i, 0))),
        out_shape=(jax.ShapeDtypeStruct((N_PAD, DW), jnp.float32),
                   jax.ShapeDtypeStruct((N_PAD, D), jnp.float32)),
    )(a0, a1, b, wl, wr)


def _fin_body(a0_ref, a1_ref, b_ref, o_ref):
    v = a0_ref[...] + a1_ref[...]
    o_ref[...] = jnp.maximum(v[:, :D] / (v[:, D:D + 1] + 1e-16) + b_ref[...], 0.0)


def _fin(a0, a1, b):
    blk = 1000
    return pl.pallas_call(
        _fin_body,
        grid=(N_NODES // blk,),
        in_specs=[pl.BlockSpec((blk, DW), lambda i: (i, 0)),
                  pl.BlockSpec((blk, DW), lambda i: (i, 0)),
                  pl.BlockSpec((D,), lambda i: (0,))],
        out_specs=pl.BlockSpec((blk, D), lambda i: (i, 0)),
        out_shape=jax.ShapeDtypeStruct((N_NODES, D), jnp.float32),
    )(a0, a1, b)


def _sc_edge_body(xl_hbm, xr_hbm, att_hbm, idx_hbm, out_hbm,
                  idxg0, idxg1, att_v, bufa0, bufa1, bufb0, bufb1, acc_sh,
                  sga0, sga1, sgb0, sgb1, ss0, ss1, si0, si1):
    cid = lax.axis_index("c")
    sid = lax.axis_index("s")
    wid = cid * NS + sid
    idxg = (idxg0, idxg1)
    bufa = (bufa0, bufa1)
    bufb = (bufb0, bufb1)
    sga = (sga0, sga1)
    sgb = (sgb0, sgb1)
    ss = (ss0, ss1)
    si = (si0, si1)

    def zrow(r, c):
        for j in range(DW // 16):
            bufa0[r, pl.ds(j * 16, 16)] = jnp.zeros((16,), jnp.float32)
        return c
    lax.fori_loop(0, CHUNK, zrow, 0)
    rows_per_tile = N_PAD // NS
    for k in range(rows_per_tile // CHUNK):
        pltpu.sync_copy(bufa0, acc_sh.at[pl.ds(sid * rows_per_tile + k * CHUNK, CHUNK)])

    pltpu.sync_copy(att_hbm, att_v)
    att_c = [att_v[pl.ds(j * 16, 16)] for j in range(8)]
    e0 = jnp.where(lax.iota(jnp.int32, 16) == 0, 1.0, 0.0)
    plsc.subcore_barrier()

    def gather_start(pb, k, st):
        pltpu.async_copy(xl_hbm.at[idxg[pb].at[0, k]], bufa[st], sga[st])
        pltpu.async_copy(xr_hbm.at[idxg[pb].at[1, k]], bufb[st], sgb[st])

    def gather_wait(pb, k, st):
        pltpu.make_async_copy(xl_hbm.at[idxg[pb].at[0, k]], bufa[st], sga[st]).wait()
        pltpu.make_async_copy(xr_hbm.at[idxg[pb].at[1, k]], bufb[st], sgb[st]).wait()

    def scat_start(pb, k, st):
        pltpu.async_copy(bufa[st], acc_sh.at[idxg[pb].at[1, k]], ss[st], add=True)

    def scat_wait(pb, k, st):
        pltpu.make_async_copy(bufa[st], acc_sh.at[idxg[pb].at[1, k]], ss[st]).wait()

    def compute(ba, bb):
        @plsc.parallel_loop(0, CHUNK, 1, unroll=4)
        def edge(e):
            a = [ba[e, pl.ds(j * 16, 16)] for j in range(8)]
            p = []
            for j in range(8):
                s = a[j] + bb[e, pl.ds(j * 16, 16)]
                p.append(att_c[j] * jnp.maximum(s, 0.2 * s))
            q = [p[0] + p[1], p[2] + p[3], p[4] + p[5], p[6] + p[7]]
            acc = (q[0] + q[1]) + (q[2] + q[3])
            ex = jnp.exp(lax.broadcast(jnp.sum(acc), (16,)))
            for j in range(8):
                ba[e, pl.ds(j * 16, 16)] = ex * a[j]
            ba[e, pl.ds(D, 16)] = ex * e0

    pltpu.sync_copy(idx_hbm.at[wid, 0], idxg[0])
    gather_start(0, 0, 0)

    def super_step(s, c):
        for q in (0, 1):
            for k in range(G):
                st = k % 2
                ot = 1 - st
                gather_wait(q, k, st)
                compute(bufa[st], bufb[st])
                scat_start(q, k, st)
                if q == 0 and k == 0:
                    @pl.when(s > 0)
                    def _():
                        scat_wait(1, G - 1, 1)
                elif k == 0:
                    scat_wait(0, G - 1, 1)
                else:
                    scat_wait(q, k - 1, ot)
                if k == 0:
                    if q == 0:
                        pltpu.async_copy(idx_hbm.at[wid, 2 * s + 1], idxg[1], si[1])
                    else:
                        @pl.when(s < SG - 1)
                        def _():
                            pltpu.async_copy(idx_hbm.at[wid, 2 * s + 2], idxg[0], si[0])
                if k < G - 1:
                    gather_start(q, k + 1, ot)
                else:
                    def _boundary(qq, ss_idx):
                        pltpu.make_async_copy(idx_hbm.at[wid, ss_idx],
                                              idxg[1 - qq], si[1 - qq]).wait()
                        gather_start(1 - qq, 0, 0)
                    if q == 0:
                        _boundary(0, 2 * s + 1)
                    else:
                        @pl.when(s < SG - 1)
                        def _():
                            _boundary(1, 2 * s + 2)
        return c
    lax.fori_loop(0, SG, super_step, 0)
    scat_wait(1, G - 1, 1)

    plsc.subcore_barrier()
    pltpu.sync_copy(acc_sh.at[pl.ds(sid * rows_per_tile, rows_per_tile)],
                    out_hbm.at[cid, sid])


@functools.cache
def _make_sc_edge():
    mesh = plsc.VectorSubcoreMesh(core_axis_name="c", subcore_axis_name="s")
    return pl.kernel(
        _sc_edge_body,
        out_type=jax.ShapeDtypeStruct((NC, NS, N_PAD // NS, DW), jnp.float32),
        mesh=mesh,
        scratch_types=[
            pltpu.VMEM((2, G, CHUNK), jnp.int32),
            pltpu.VMEM((2, G, CHUNK), jnp.int32),
            pltpu.VMEM((D,), jnp.float32),
            pltpu.VMEM((CHUNK, DW), jnp.float32),
            pltpu.VMEM((CHUNK, DW), jnp.float32),
            pltpu.VMEM((CHUNK, D), jnp.float32),
            pltpu.VMEM((CHUNK, D), jnp.float32),
            pltpu.VMEM_SHARED((N_PAD, DW), jnp.float32),
            pltpu.SemaphoreType.DMA,
            pltpu.SemaphoreType.DMA,
            pltpu.SemaphoreType.DMA,
            pltpu.SemaphoreType.DMA,
            pltpu.SemaphoreType.DMA,
            pltpu.SemaphoreType.DMA,
            pltpu.SemaphoreType.DMA,
            pltpu.SemaphoreType.DMA,
        ],
        compiler_params=pltpu.CompilerParams(use_tc_tiling_on_sc=False,
                                             needs_layout_passes=False),
    )


def _sc_edge(xl, xr, att, idx):
    acc = _make_sc_edge()(xl, xr, att, idx)
    return jnp.reshape(acc, (NC, N_PAD, DW))


def kernel(node_features, Wl1, Wr1, att1, b1, Wl2, Wr2, att2, b2, edge_index):
    x0 = jnp.pad(node_features, ((0, N_PAD - N_NODES), (0, 0)))
    ei = edge_index.astype(jnp.int32)
    loop = jnp.arange(N_NODES, dtype=jnp.int32)
    pad = jnp.full((E_PAD - E_TOT,), TRASH, jnp.int32)
    src = jnp.concatenate([ei[0], loop, pad]).reshape(NW, NGRP, G, CHUNK)
    dst = jnp.concatenate([ei[1], loop, pad]).reshape(NW, NGRP, G, CHUNK)
    idx = jnp.stack([src, dst], axis=2)

    xl1, xr1 = _mm2(x0, Wl1, Wr1)
    acc1 = _sc_edge(xl1, xr1, att1, idx)
    xl2, xr2 = _mid(acc1[0], acc1[1], b1, Wl2, Wr2)
    acc2 = _sc_edge(xl2, xr2, att2, idx)
    return _fin(acc2[0], acc2[1], b2)
